# baseline (speedup 1.0000x reference)
import functools

import jax
import jax.numpy as jnp
from jax.experimental import pallas as pl
from jax.experimental.pallas import tpu as pltpu
from jax.experimental.pallas import tpu_sc as plsc

TM = 128
BH = 256


def _routing_body(
    x_ref, gw_ref, pos0_ref, pos1_ref, w0_ref, w1_ref,
    ti_ref, ei_ref, tot_ref,
):
    x = x_ref[...]
    gw = gw_ref[...]
    logits = jax.lax.dot_general(
        x, gw, (((1,), (1,)), ((), ())), preferred_element_type=jnp.float32
    )
    t, e = logits.shape
    neg = jnp.float32(-1e30)
    eidx = jax.lax.broadcasted_iota(jnp.int32, (t, e), 1)
    m0 = jnp.max(logits, axis=1, keepdims=True)
    a0 = jnp.min(jnp.where(logits == m0, eidx, e), axis=1, keepdims=True)
    oh0 = eidx == a0
    masked = jnp.where(oh0, neg, logits)
    m1 = jnp.max(masked, axis=1, keepdims=True)
    a1 = jnp.min(jnp.where(masked == m1, eidx, e), axis=1, keepdims=True)
    oh1 = eidx == a1
    z = jnp.exp(m1 - m0)
    denom = 1.0 + z
    w0_ref[...] = (1.0 / denom).astype(jnp.float32)
    w1_ref[...] = (z / denom).astype(jnp.float32)
    both = oh0.astype(jnp.float32) + oh1.astype(jnp.float32)
    blk = 128
    ridx = jax.lax.broadcasted_iota(jnp.int32, (blk, t), 1)
    excl_blocks = []
    for c in range(t // blk):
        rows = c * blk + jax.lax.broadcasted_iota(jnp.int32, (blk, t), 0)
        tri = (ridx < rows).astype(jnp.float32)
        excl_blocks.append(
            jax.lax.dot_general(
                tri, both, (((1,), (0,)), ((), ())),
                preferred_element_type=jnp.float32,
            )
        )
    excl = jnp.concatenate(excl_blocks, axis=0)
    counts = jnp.sum(both, axis=0, keepdims=True)
    ce = jax.lax.broadcasted_iota(jnp.int32, (e, e), 0)
    re_ = jax.lax.broadcasted_iota(jnp.int32, (e, e), 1)
    tril_e = (ce < re_).astype(jnp.float32)
    inv_tm = jnp.float32(1.0 / TM)
    ftm = jnp.float32(TM)
    cpad = jnp.floor((counts + (TM - 1.0)) * inv_tm) * ftm
    starts = jax.lax.dot_general(
        cpad, tril_e, (((1,), (0,)), ((), ())),
        preferred_element_type=jnp.float32,
    )
    base = starts + excl
    pos0 = jnp.sum(jnp.where(oh0, base, 0.0), axis=1)
    pos1 = jnp.sum(jnp.where(oh1, base, 0.0), axis=1)
    pos0_ref[...] = pos0[:, None].astype(jnp.int32)
    pos1_ref[...] = pos1[:, None].astype(jnp.int32)

    wpad = ti_ref.shape[0]
    nt = cpad * inv_tm
    us = jax.lax.dot_general(
        nt, tril_e, (((1,), (0,)), ((), ())),
        preferred_element_type=jnp.float32,
    )
    total = jnp.sum(nt)
    first = starts * inv_tm
    uvec = jax.lax.broadcasted_iota(jnp.int32, (wpad, 1), 0).astype(jnp.float32)
    cmp = us <= uvec
    e_of = jnp.sum(cmp.astype(jnp.float32), axis=1, keepdims=True) - 1.0
    eidw = jax.lax.broadcasted_iota(jnp.int32, (wpad, e), 1).astype(jnp.float32)
    ohw = (eidw == e_of).astype(jnp.float32)
    first_u = jnp.sum(ohw * first, axis=1, keepdims=True)
    us_u = jnp.sum(ohw * us, axis=1, keepdims=True)
    ti_u = first_u + (uvec - us_u)
    valid = uvec < total
    lastmask = (uvec == total - 1.0).astype(jnp.float32)
    ti_last = jnp.sum(lastmask * ti_u)
    ei_last = jnp.sum(lastmask * e_of)
    ti_u = jnp.where(valid, ti_u, ti_last)
    e_of = jnp.where(valid, e_of, ei_last)
    ti_ref[...] = ti_u.astype(jnp.int32)
    ei_ref[...] = e_of.astype(jnp.int32)
    tot_ref[...] = (jnp.zeros((1, 1), jnp.float32) + total).astype(jnp.int32)


def _routing(x, gate_w, wpad):
    t, _ = x.shape
    e = gate_w.shape[0]
    return pl.pallas_call(
        _routing_body,
        out_shape=(
            jax.ShapeDtypeStruct((t, 1), jnp.int32),
            jax.ShapeDtypeStruct((t, 1), jnp.int32),
            jax.ShapeDtypeStruct((t, 1), jnp.float32),
            jax.ShapeDtypeStruct((t, 1), jnp.float32),
            jax.ShapeDtypeStruct((wpad, 1), jnp.int32),
            jax.ShapeDtypeStruct((wpad, 1), jnp.int32),
            jax.ShapeDtypeStruct((1, 1), jnp.int32),
        ),
    )(x, gate_w)


def _gmm_inner_body(idxs, xs_b, w1_b, w3_b, w2_b, out_b):
    _, hj = idxs
    x_t = xs_b[...]
    w1c = w1_b[0]
    w3c = w3_b[0]
    w2c = w2_b[0]
    xw1 = jax.lax.dot_general(
        x_t, w1c, (((1,), (1,)), ((), ())), preferred_element_type=jnp.float32
    )
    xw3 = jax.lax.dot_general(
        x_t, w3c, (((1,), (1,)), ((), ())), preferred_element_type=jnp.float32
    )
    h = (xw1 * jax.nn.sigmoid(xw1)) * xw3
    o = jax.lax.dot_general(
        h, w2c, (((1,), (0,)), ((), ())), preferred_element_type=jnp.float32
    )
    @pl.when(hj == 0)
    def _():
        out_b[...] = o

    @pl.when(hj != 0)
    def _():
        out_b[...] = out_b[...] + o


def _make_gmm_outer(d, nhj):
    def _gmm_outer_body(
        ti_ref, ei_ref, tot_ref, xs_hbm, w1_hbm, w2_hbm, w3_hbm, out_hbm
    ):
        total = tot_ref[0]
        wmode = pl.Buffered(buffer_count=3, use_lookahead=True)
        wspec1 = pl.BlockSpec(
            (1, BH, d), lambda u, hj: (ei_ref[u], hj, 0), pipeline_mode=wmode)
        wspec3 = pl.BlockSpec(
            (1, BH, d), lambda u, hj: (ei_ref[u], hj, 0), pipeline_mode=wmode)
        wspec2 = pl.BlockSpec(
            (1, BH, d), lambda u, hj: (ei_ref[u], hj, 0), pipeline_mode=wmode)
        xspec = pl.BlockSpec(
            (TM, d), lambda u, hj: (ti_ref[u], 0),
            pipeline_mode=pl.Buffered(buffer_count=3, use_lookahead=True))
        ospec = pl.BlockSpec((TM, d), lambda u, hj: (ti_ref[u], 0))
        pltpu.emit_pipeline(
            _gmm_inner_body,
            grid=(total, nhj),
            in_specs=[xspec, wspec1, wspec3, wspec2],
            out_specs=[ospec],
            _explicit_indices=True,
        )(xs_hbm, w1_hbm, w3_hbm, w2_hbm, out_hbm)

    return _gmm_outer_body


def _gmm(xs, w1, w2, w3, ti, ei, tot):
    sp, d = xs.shape
    e, h, _ = w1.shape
    nhj = h // BH
    anyspec = pl.BlockSpec(memory_space=pl.ANY)
    smemspec = pl.BlockSpec(memory_space=pltpu.SMEM)
    return pl.pallas_call(
        _make_gmm_outer(d, nhj),
        in_specs=[smemspec, smemspec, smemspec,
                  anyspec, anyspec, anyspec, anyspec],
        out_specs=anyspec,
        out_shape=jax.ShapeDtypeStruct((sp, d), jnp.float32),
    )(ti, ei, tot, xs, w1, w2, w3)


def _combine_body(g0_ref, g1_ref, w0_ref, w1_ref, y_ref):
    y_ref[...] = g0_ref[...] * w0_ref[...] + g1_ref[...] * w1_ref[...]


def _combine(g0, g1, w0, w1):
    t, d = g0.shape
    tmc = 256
    return pl.pallas_call(
        _combine_body,
        grid=(t // tmc,),
        in_specs=[
            pl.BlockSpec((tmc, d), lambda i: (i, 0)),
            pl.BlockSpec((tmc, d), lambda i: (i, 0)),
            pl.BlockSpec((tmc, 1), lambda i: (i, 0)),
            pl.BlockSpec((tmc, 1), lambda i: (i, 0)),
        ],
        out_specs=pl.BlockSpec((tmc, d), lambda i: (i, 0)),
        out_shape=jax.ShapeDtypeStruct((t, d), jnp.float32),
    )(g0, g1, w0, w1)


_NC = 2
_NS = 16
_NW = _NC * _NS


def _sc_dispatch(x, p0, p1, s):
    t, d = x.shape
    bpw = t // _NW
    mesh = plsc.VectorSubcoreMesh(core_axis_name="c", subcore_axis_name="s")

    @functools.partial(
        pl.kernel,
        mesh=mesh,
        out_type=jax.ShapeDtypeStruct((s, d), jnp.float32),
        scratch_types=[
            pltpu.VMEM((bpw,), jnp.int32),
            pltpu.VMEM((bpw,), jnp.int32),
            pltpu.VMEM((bpw, d), jnp.float32),
            pltpu.SemaphoreType.DMA,
        ],
    )
    def k(x_hbm, p0_hbm, p1_hbm, o_hbm, i0_v, i1_v, rows_v, sem):
        wid = jax.lax.axis_index("s") * _NC + jax.lax.axis_index("c")
        base = wid * bpw
        pltpu.sync_copy(p0_hbm.at[pl.ds(base, bpw)], i0_v)
        pltpu.sync_copy(p1_hbm.at[pl.ds(base, bpw)], i1_v)
        pltpu.sync_copy(x_hbm.at[pl.ds(base, bpw)], rows_v)
        pltpu.async_copy(rows_v, o_hbm.at[i0_v], sem).wait()
        pltpu.async_copy(rows_v, o_hbm.at[i1_v], sem).wait()

    return k(x, p0, p1)


def _sc_gather2(os_, p0, p1):
    s, d = os_.shape
    t = p0.shape[0]
    bpw = t // _NW
    mesh = plsc.VectorSubcoreMesh(core_axis_name="c", subcore_axis_name="s")
    ot = jax.ShapeDtypeStruct((t, d), jnp.float32)

    @functools.partial(
        pl.kernel,
        mesh=mesh,
        out_type=(ot, ot),
        scratch_types=[
            pltpu.VMEM((bpw,), jnp.int32),
            pltpu.VMEM((bpw, d), jnp.float32),
            pltpu.SemaphoreType.DMA,
        ],
    )
    def k(os_hbm, p0_hbm, p1_hbm, g0_hbm, g1_hbm, idx_v, rows_v, sem):
        wid = jax.lax.axis_index("s") * _NC + jax.lax.axis_index("c")
        base = wid * bpw
        pltpu.sync_copy(p0_hbm.at[pl.ds(base, bpw)], idx_v)
        pltpu.async_copy(os_hbm.at[idx_v], rows_v, sem).wait()
        pltpu.sync_copy(rows_v, g0_hbm.at[pl.ds(base, bpw)])
        pltpu.sync_copy(p1_hbm.at[pl.ds(base, bpw)], idx_v)
        pltpu.async_copy(os_hbm.at[idx_v], rows_v, sem).wait()
        pltpu.sync_copy(rows_v, g1_hbm.at[pl.ds(base, bpw)])

    return k(os_, p0, p1)


def kernel(x, gate_w, w1, w2, w3):
    t, d = x.shape
    e = gate_w.shape[0]
    k = 2
    s = t * k
    w_static = s // TM + e
    spad = s + e * TM

    wpad = -(-w_static // 8) * 8

    pos0, pos1, wt0, wt1, ti2, ei2, tot2 = _routing(x, gate_w, wpad)
    p0r = pos0.reshape(t)
    p1r = pos1.reshape(t)
    ti = ti2.reshape(wpad)
    ei = ei2.reshape(wpad)
    tot = tot2.reshape(1)

    xs = _sc_dispatch(x, p0r, p1r, spad)
    os = _gmm(xs, w1, w2, w3, ti, ei, tot)
    g0, g1 = _sc_gather2(os, p0r, p1r)
    return _combine(g0, g1, wt0, wt1)


# speedup vs baseline: 1.0639x; 1.0639x over previous
import functools

import jax
import jax.numpy as jnp
from jax.experimental import pallas as pl
from jax.experimental.pallas import tpu as pltpu
from jax.experimental.pallas import tpu_sc as plsc

TM = 128
BH = 512


def _routing_body(
    x_ref, gw_ref, pos0_ref, pos1_ref, w0_ref, w1_ref,
    ti_ref, ei_ref, tot_ref,
):
    x = x_ref[...]
    gw = gw_ref[...]
    logits = jax.lax.dot_general(
        x, gw, (((1,), (1,)), ((), ())), preferred_element_type=jnp.float32
    )
    t, e = logits.shape
    neg = jnp.float32(-1e30)
    eidx = jax.lax.broadcasted_iota(jnp.int32, (t, e), 1)
    m0 = jnp.max(logits, axis=1, keepdims=True)
    a0 = jnp.min(jnp.where(logits == m0, eidx, e), axis=1, keepdims=True)
    oh0 = eidx == a0
    masked = jnp.where(oh0, neg, logits)
    m1 = jnp.max(masked, axis=1, keepdims=True)
    a1 = jnp.min(jnp.where(masked == m1, eidx, e), axis=1, keepdims=True)
    oh1 = eidx == a1
    z = jnp.exp(m1 - m0)
    denom = 1.0 + z
    w0_ref[...] = (1.0 / denom).astype(jnp.float32)
    w1_ref[...] = (z / denom).astype(jnp.float32)
    both = oh0.astype(jnp.float32) + oh1.astype(jnp.float32)
    blk = 128
    ridx = jax.lax.broadcasted_iota(jnp.int32, (blk, t), 1)
    excl_blocks = []
    for c in range(t // blk):
        rows = c * blk + jax.lax.broadcasted_iota(jnp.int32, (blk, t), 0)
        tri = (ridx < rows).astype(jnp.float32)
        excl_blocks.append(
            jax.lax.dot_general(
                tri, both, (((1,), (0,)), ((), ())),
                preferred_element_type=jnp.float32,
            )
        )
    excl = jnp.concatenate(excl_blocks, axis=0)
    counts = jnp.sum(both, axis=0, keepdims=True)
    ce = jax.lax.broadcasted_iota(jnp.int32, (e, e), 0)
    re_ = jax.lax.broadcasted_iota(jnp.int32, (e, e), 1)
    tril_e = (ce < re_).astype(jnp.float32)
    inv_tm = jnp.float32(1.0 / TM)
    ftm = jnp.float32(TM)
    cpad = jnp.floor((counts + (TM - 1.0)) * inv_tm) * ftm
    starts = jax.lax.dot_general(
        cpad, tril_e, (((1,), (0,)), ((), ())),
        preferred_element_type=jnp.float32,
    )
    base = starts + excl
    pos0 = jnp.sum(jnp.where(oh0, base, 0.0), axis=1)
    pos1 = jnp.sum(jnp.where(oh1, base, 0.0), axis=1)
    pos0_ref[...] = pos0[:, None].astype(jnp.int32)
    pos1_ref[...] = pos1[:, None].astype(jnp.int32)

    wpad = ti_ref.shape[0]
    nt = cpad * inv_tm
    us = jax.lax.dot_general(
        nt, tril_e, (((1,), (0,)), ((), ())),
        preferred_element_type=jnp.float32,
    )
    total = jnp.sum(nt)
    first = starts * inv_tm
    uvec = jax.lax.broadcasted_iota(jnp.int32, (wpad, 1), 0).astype(jnp.float32)
    cmp = us <= uvec
    e_of = jnp.sum(cmp.astype(jnp.float32), axis=1, keepdims=True) - 1.0
    eidw = jax.lax.broadcasted_iota(jnp.int32, (wpad, e), 1).astype(jnp.float32)
    ohw = (eidw == e_of).astype(jnp.float32)
    first_u = jnp.sum(ohw * first, axis=1, keepdims=True)
    us_u = jnp.sum(ohw * us, axis=1, keepdims=True)
    ti_u = first_u + (uvec - us_u)
    valid = uvec < total
    lastmask = (uvec == total - 1.0).astype(jnp.float32)
    ti_last = jnp.sum(lastmask * ti_u)
    ei_last = jnp.sum(lastmask * e_of)
    ti_u = jnp.where(valid, ti_u, ti_last)
    e_of = jnp.where(valid, e_of, ei_last)
    ti_ref[...] = ti_u.astype(jnp.int32)
    ei_ref[...] = e_of.astype(jnp.int32)
    tot_ref[...] = (jnp.zeros((1, 1), jnp.float32) + total).astype(jnp.int32)


def _routing(x, gate_w, wpad):
    t, _ = x.shape
    e = gate_w.shape[0]
    return pl.pallas_call(
        _routing_body,
        out_shape=(
            jax.ShapeDtypeStruct((t, 1), jnp.int32),
            jax.ShapeDtypeStruct((t, 1), jnp.int32),
            jax.ShapeDtypeStruct((t, 1), jnp.float32),
            jax.ShapeDtypeStruct((t, 1), jnp.float32),
            jax.ShapeDtypeStruct((wpad, 1), jnp.int32),
            jax.ShapeDtypeStruct((wpad, 1), jnp.int32),
            jax.ShapeDtypeStruct((1, 1), jnp.int32),
        ),
    )(x, gate_w)


def _gmm_inner_body(idxs, xs_b, w1_b, w3_b, w2_b, out_b):
    _, hj = idxs
    x_t = xs_b[...]
    w1c = w1_b[0]
    w3c = w3_b[0]
    w2c = w2_b[0]
    xw1 = jax.lax.dot_general(
        x_t, w1c, (((1,), (1,)), ((), ())), preferred_element_type=jnp.float32
    )
    xw3 = jax.lax.dot_general(
        x_t, w3c, (((1,), (1,)), ((), ())), preferred_element_type=jnp.float32
    )
    h = (xw1 * jax.nn.sigmoid(xw1)) * xw3
    o = jax.lax.dot_general(
        h, w2c, (((1,), (0,)), ((), ())), preferred_element_type=jnp.float32
    )
    @pl.when(hj == 0)
    def _():
        out_b[...] = o

    @pl.when(hj != 0)
    def _():
        out_b[...] = out_b[...] + o


def _make_gmm_outer(d, nhj):
    def _gmm_outer_body(
        ti_ref, ei_ref, tot_ref, xs_hbm, w1_hbm, w2_hbm, w3_hbm, out_hbm
    ):
        total = tot_ref[0]
        wmode = pl.Buffered(buffer_count=3, use_lookahead=True)
        wspec1 = pl.BlockSpec(
            (1, BH, d), lambda u, hj: (ei_ref[u], hj, 0), pipeline_mode=wmode)
        wspec3 = pl.BlockSpec(
            (1, BH, d), lambda u, hj: (ei_ref[u], hj, 0), pipeline_mode=wmode)
        wspec2 = pl.BlockSpec(
            (1, BH, d), lambda u, hj: (ei_ref[u], hj, 0), pipeline_mode=wmode)
        xspec = pl.BlockSpec(
            (TM, d), lambda u, hj: (ti_ref[u], 0),
            pipeline_mode=pl.Buffered(buffer_count=3, use_lookahead=True))
        ospec = pl.BlockSpec((TM, d), lambda u, hj: (ti_ref[u], 0))
        pltpu.emit_pipeline(
            _gmm_inner_body,
            grid=(total, nhj),
            in_specs=[xspec, wspec1, wspec3, wspec2],
            out_specs=[ospec],
            _explicit_indices=True,
        )(xs_hbm, w1_hbm, w3_hbm, w2_hbm, out_hbm)

    return _gmm_outer_body


def _gmm(xs, w1, w2, w3, ti, ei, tot):
    sp, d = xs.shape
    e, h, _ = w1.shape
    nhj = h // BH
    anyspec = pl.BlockSpec(memory_space=pl.ANY)
    smemspec = pl.BlockSpec(memory_space=pltpu.SMEM)
    return pl.pallas_call(
        _make_gmm_outer(d, nhj),
        in_specs=[smemspec, smemspec, smemspec,
                  anyspec, anyspec, anyspec, anyspec],
        out_specs=anyspec,
        out_shape=jax.ShapeDtypeStruct((sp, d), jnp.float32),
    )(ti, ei, tot, xs, w1, w2, w3)


def _combine_body(g0_ref, g1_ref, w0_ref, w1_ref, y_ref):
    y_ref[...] = g0_ref[...] * w0_ref[...] + g1_ref[...] * w1_ref[...]


def _combine(g0, g1, w0, w1):
    t, d = g0.shape
    tmc = 256
    return pl.pallas_call(
        _combine_body,
        grid=(t // tmc,),
        in_specs=[
            pl.BlockSpec((tmc, d), lambda i: (i, 0)),
            pl.BlockSpec((tmc, d), lambda i: (i, 0)),
            pl.BlockSpec((tmc, 1), lambda i: (i, 0)),
            pl.BlockSpec((tmc, 1), lambda i: (i, 0)),
        ],
        out_specs=pl.BlockSpec((tmc, d), lambda i: (i, 0)),
        out_shape=jax.ShapeDtypeStruct((t, d), jnp.float32),
    )(g0, g1, w0, w1)


_NC = 2
_NS = 16
_NW = _NC * _NS


def _sc_dispatch(x, p0, p1, s):
    t, d = x.shape
    bpw = t // _NW
    mesh = plsc.VectorSubcoreMesh(core_axis_name="c", subcore_axis_name="s")

    @functools.partial(
        pl.kernel,
        mesh=mesh,
        out_type=jax.ShapeDtypeStruct((s, d), jnp.float32),
        scratch_types=[
            pltpu.VMEM((bpw,), jnp.int32),
            pltpu.VMEM((bpw,), jnp.int32),
            pltpu.VMEM((bpw, d), jnp.float32),
            pltpu.SemaphoreType.DMA,
        ],
    )
    def k(x_hbm, p0_hbm, p1_hbm, o_hbm, i0_v, i1_v, rows_v, sem):
        wid = jax.lax.axis_index("s") * _NC + jax.lax.axis_index("c")
        base = wid * bpw
        pltpu.sync_copy(p0_hbm.at[pl.ds(base, bpw)], i0_v)
        pltpu.sync_copy(p1_hbm.at[pl.ds(base, bpw)], i1_v)
        pltpu.sync_copy(x_hbm.at[pl.ds(base, bpw)], rows_v)
        pltpu.async_copy(rows_v, o_hbm.at[i0_v], sem).wait()
        pltpu.async_copy(rows_v, o_hbm.at[i1_v], sem).wait()

    return k(x, p0, p1)


def _sc_gather2(os_, p0, p1):
    s, d = os_.shape
    t = p0.shape[0]
    bpw = t // _NW
    mesh = plsc.VectorSubcoreMesh(core_axis_name="c", subcore_axis_name="s")
    ot = jax.ShapeDtypeStruct((t, d), jnp.float32)

    @functools.partial(
        pl.kernel,
        mesh=mesh,
        out_type=(ot, ot),
        scratch_types=[
            pltpu.VMEM((bpw,), jnp.int32),
            pltpu.VMEM((bpw, d), jnp.float32),
            pltpu.SemaphoreType.DMA,
        ],
    )
    def k(os_hbm, p0_hbm, p1_hbm, g0_hbm, g1_hbm, idx_v, rows_v, sem):
        wid = jax.lax.axis_index("s") * _NC + jax.lax.axis_index("c")
        base = wid * bpw
        pltpu.sync_copy(p0_hbm.at[pl.ds(base, bpw)], idx_v)
        pltpu.async_copy(os_hbm.at[idx_v], rows_v, sem).wait()
        pltpu.sync_copy(rows_v, g0_hbm.at[pl.ds(base, bpw)])
        pltpu.sync_copy(p1_hbm.at[pl.ds(base, bpw)], idx_v)
        pltpu.async_copy(os_hbm.at[idx_v], rows_v, sem).wait()
        pltpu.sync_copy(rows_v, g1_hbm.at[pl.ds(base, bpw)])

    return k(os_, p0, p1)


def kernel(x, gate_w, w1, w2, w3):
    t, d = x.shape
    e = gate_w.shape[0]
    k = 2
    s = t * k
    w_static = s // TM + e
    spad = s + e * TM

    wpad = -(-w_static // 8) * 8

    pos0, pos1, wt0, wt1, ti2, ei2, tot2 = _routing(x, gate_w, wpad)
    p0r = pos0.reshape(t)
    p1r = pos1.reshape(t)
    ti = ti2.reshape(wpad)
    ei = ei2.reshape(wpad)
    tot = tot2.reshape(1)

    xs = _sc_dispatch(x, p0r, p1r, spad)
    os = _gmm(xs, w1, w2, w3, ti, ei, tot)
    g0, g1 = _sc_gather2(os, p0r, p1r)
    return _combine(g0, g1, wt0, wt1)
